# baseline (speedup 1.0000x reference)
import jax
import jax.numpy as jnp
from jax import lax
from jax.experimental import pallas as pl
from jax.experimental.pallas import tpu as pltpu, tpu_sc as plsc

NC = 2
NS = 16
NW = NC * NS

BT = 16384
S = 20
D = 64
N_TOK = BT * S

DP = 128
CHUNK = 128
N_CHUNKS = N_TOK // CHUNK
CH_PER_W = N_CHUNKS // NW
NBUF = 4
NGRP = CH_PER_W // NBUF


def _emb_body(table_hbm, tok_hbm, out_hbm, idx_v, gbuf, gsems, wsems):
    wid = lax.axis_index("s") * NC + lax.axis_index("c")
    c0 = wid * CH_PER_W

    pltpu.sync_copy(tok_hbm.at[pl.ds(c0, CH_PER_W)], idx_v)

    def gather(j, b):
        return pltpu.make_async_copy(
            table_hbm.at[idx_v.at[j]], gbuf.at[b], gsems.at[b])

    def wout(j, b):
        return pltpu.make_async_copy(
            gbuf.at[b],
            out_hbm.at[pl.ds((c0 + j) * CHUNK, CHUNK)],
            wsems.at[b])

    for b in range(NBUF):
        gather(b, b).start()

    @pl.loop(0, NGRP - 1)
    def _(g):
        j0 = g * NBUF
        for b in range(NBUF):
            j = j0 + b
            gather(j, b).wait()
            wout(j, b).start()
            wout(j, b).wait()
            gather(j + NBUF, b).start()

    for b in range(NBUF):
        j = CH_PER_W - NBUF + b
        gather(j, b).wait()
        wout(j, b).start()
    for b in range(NBUF):
        wout(CH_PER_W - NBUF + b, b).wait()


@jax.jit
def _emb_lookup(tok2d, table):
    mesh = plsc.VectorSubcoreMesh(core_axis_name="c", subcore_axis_name="s")
    run = pl.kernel(
        _emb_body,
        out_type=jax.ShapeDtypeStruct((N_TOK, DP), jnp.float32),
        mesh=mesh,
        scratch_types=[
            pltpu.VMEM((CH_PER_W, CHUNK), jnp.int32),
            pltpu.VMEM((NBUF, CHUNK, DP), jnp.float32),
            pltpu.SemaphoreType.DMA((NBUF,)),
            pltpu.SemaphoreType.DMA((NBUF,)),
        ],
    )
    return run(table, tok2d)


def kernel(tokens, emb_weight):
    tok2d = tokens.reshape(N_CHUNKS, CHUNK).astype(jnp.int32)
    table_pad = jnp.pad(emb_weight.astype(jnp.float32), ((0, 0), (0, DP - D)))
    out = _emb_lookup(tok2d, table_pad)
    return out[:, :D].reshape(BT, S, D)
